# baseline (speedup 1.0000x reference)
import jax
import jax.numpy as jnp
from jax.experimental import pallas as pl
from jax.experimental.pallas import tpu as pltpu

_B = 8192
_IN = 16
_CTX = 384
_KD = 64
_NS = 512
_NSUB = 8
_HH = 128
_HO = 65
_BB = 1024


def _gelu(v):
    return 0.5 * v * (1.0 + jax.lax.erf(v * 0.7071067811865476))


def _fused(x_ref, ctx_ref, s2s_ref, keys_ref, qW1_ref, qb1_ref, qW2_ref,
           qb2_ref, hW1_ref, hb1_ref, hW2_ref, hb2_ref, temp_ref, out_ref):
    f32 = jnp.float32
    xb = x_ref[...]
    ctxb = ctx_ref[...]

    t = jnp.dot(xb, qW1_ref[...], preferred_element_type=f32) + qb1_ref[...]
    t = _gelu(t)
    q = jnp.dot(t, qW2_ref[...], preferred_element_type=f32) + qb2_ref[...]
    q = q * (1.0 / temp_ref[0, 0])

    al = jax.lax.dot_general(q, keys_ref[...], (((1,), (1,)), ((), ())),
                             preferred_element_type=f32)
    al = al - jnp.max(al, axis=-1, keepdims=True)
    ea = jnp.exp(al)
    r = jnp.dot(ea, s2s_ref[...], preferred_element_type=f32)
    sl = r / jnp.sum(ea, axis=-1, keepdims=True)
    sl = sl - jnp.max(sl, axis=-1, keepdims=True)
    es = jnp.exp(sl)
    p = es / jnp.sum(es, axis=-1, keepdims=True)

    acc = jnp.dot(p, hb2_ref[...], preferred_element_type=f32)
    for e in range(_NSUB):
        he = (jnp.dot(xb, hW1_ref[e, :_IN, :], preferred_element_type=f32)
              + jnp.dot(ctxb, hW1_ref[e, _IN:, :], preferred_element_type=f32)
              + hb1_ref[e:e + 1, :])
        he = _gelu(he)
        oe = jnp.dot(he, hW2_ref[e], preferred_element_type=f32)
        acc = acc + p[:, e:e + 1] * oe
    out_ref[...] = acc


def kernel(x, ctx, sys2sub, keys_p, qW1, qb1, qW2, qb2, hW1, hb1, hW2, hb2,
           temp):
    f32 = jnp.float32
    grid = (_B // _BB,)
    tok = lambda i: (i, 0)
    rep = lambda i: (0, 0)
    rep3 = lambda i: (0, 0, 0)

    return pl.pallas_call(
        _fused,
        grid=grid,
        in_specs=[
            pl.BlockSpec((_BB, _IN), tok),
            pl.BlockSpec((_BB, _CTX), tok),
            pl.BlockSpec((_NS, _NSUB), rep),
            pl.BlockSpec((_NS, _KD), rep),
            pl.BlockSpec((_IN, _KD), rep),
            pl.BlockSpec((1, _KD), rep),
            pl.BlockSpec((_KD, _KD), rep),
            pl.BlockSpec((1, _KD), rep),
            pl.BlockSpec((_NSUB, _IN + _CTX, _HH), rep3),
            pl.BlockSpec((_NSUB, _HH), rep),
            pl.BlockSpec((_NSUB, _HH, _HO), rep3),
            pl.BlockSpec((_NSUB, _HO), rep),
            pl.BlockSpec((1, 1), rep),
        ],
        out_specs=pl.BlockSpec((_BB, _HO), tok),
        out_shape=jax.ShapeDtypeStruct((_B, _HO), f32),
        compiler_params=pltpu.CompilerParams(
            dimension_semantics=("parallel",)),
    )(x, ctx, sys2sub, keys_p, qW1, qb1.reshape(1, _KD), qW2,
      qb2.reshape(1, _KD), hW1, hb1, hW2, hb2, temp.reshape(1, 1))


# speedup vs baseline: 1.0031x; 1.0031x over previous
import jax
import jax.numpy as jnp
from jax.experimental import pallas as pl
from jax.experimental.pallas import tpu as pltpu

_B = 8192
_IN = 16
_CTX = 384
_KD = 64
_NS = 512
_NSUB = 8
_HH = 128
_HO = 65
_BB = 1024


def _gelu(v):
    return 0.5 * v * (1.0 + jax.lax.erf(v * 0.7071067811865476))


def _fused(x_ref, ctx_ref, s2s_ref, keys_ref, qW1_ref, qb1_ref, qW2_ref,
           qb2_ref, hW1_ref, hb1_ref, hW2_ref, hb2_ref, temp_ref, out_ref):
    f32 = jnp.float32
    xb = x_ref[...]
    ctxb = ctx_ref[...]

    t = (jnp.dot(xb, qW1_ref[...], preferred_element_type=f32)
         + qb1_ref[...][None, :])
    t = _gelu(t)
    q = (jnp.dot(t, qW2_ref[...], preferred_element_type=f32)
         + qb2_ref[...][None, :])
    q = q * (1.0 / temp_ref[0])

    al = jax.lax.dot_general(q, keys_ref[...], (((1,), (1,)), ((), ())),
                             preferred_element_type=f32)
    al = al - jnp.max(al, axis=-1, keepdims=True)
    ea = jnp.exp(al)
    r = jnp.dot(ea, s2s_ref[...], preferred_element_type=f32)
    sl = r / jnp.sum(ea, axis=-1, keepdims=True)
    sl = sl - jnp.max(sl, axis=-1, keepdims=True)
    es = jnp.exp(sl)
    p = es / jnp.sum(es, axis=-1, keepdims=True)

    acc = jnp.dot(p, hb2_ref[...], preferred_element_type=f32)
    for e in range(_NSUB):
        he = (jnp.dot(xb, hW1_ref[e, :_IN, :], preferred_element_type=f32)
              + jnp.dot(ctxb, hW1_ref[e, _IN:, :], preferred_element_type=f32)
              + hb1_ref[e:e + 1, :])
        he = _gelu(he)
        oe = jnp.dot(he, hW2_ref[e], preferred_element_type=f32)
        acc = acc + p[:, e:e + 1] * oe
    out_ref[...] = acc


def kernel(x, ctx, sys2sub, keys_p, qW1, qb1, qW2, qb2, hW1, hb1, hW2, hb2,
           temp):
    f32 = jnp.float32
    grid = (_B // _BB,)
    tok = lambda i: (i, 0)
    rep = lambda i: (0, 0)
    rep3 = lambda i: (0, 0, 0)

    return pl.pallas_call(
        _fused,
        grid=grid,
        in_specs=[
            pl.BlockSpec((_BB, _IN), tok),
            pl.BlockSpec((_BB, _CTX), tok),
            pl.BlockSpec((_NS, _NSUB), rep),
            pl.BlockSpec((_NS, _KD), rep),
            pl.BlockSpec((_IN, _KD), rep),
            pl.BlockSpec((_KD,), lambda i: (0,)),
            pl.BlockSpec((_KD, _KD), rep),
            pl.BlockSpec((_KD,), lambda i: (0,)),
            pl.BlockSpec((_NSUB, _IN + _CTX, _HH), rep3),
            pl.BlockSpec((_NSUB, _HH), rep),
            pl.BlockSpec((_NSUB, _HH, _HO), rep3),
            pl.BlockSpec((_NSUB, _HO), rep),
            pl.BlockSpec(memory_space=pltpu.SMEM),
        ],
        out_specs=pl.BlockSpec((_BB, _HO), tok),
        out_shape=jax.ShapeDtypeStruct((_B, _HO), f32),
        compiler_params=pltpu.CompilerParams(
            dimension_semantics=("parallel",)),
    )(x, ctx, sys2sub, keys_p, qW1, qb1, qW2, qb2, hW1, hb1, hW2, hb2,
      temp.reshape(1))


# speedup vs baseline: 1.0151x; 1.0120x over previous
import jax
import jax.numpy as jnp
from jax.experimental import pallas as pl
from jax.experimental.pallas import tpu as pltpu

_B = 8192
_IN = 16
_CTX = 384
_KD = 64
_NS = 512
_NSUB = 8
_HH = 128
_HO = 65
_BB = 1024


def _gelu(v):
    return 0.5 * v * (1.0 + jax.lax.erf(v * 0.7071067811865476))


def _fused(x_ref, ctx_ref, s2s_ref, keys_ref, qW1_ref, qb1_ref, qW2_ref,
           qb2_ref, w1_ref, hb1_ref, w2_ref, hb2_ref, temp_ref, out_ref):
    f32 = jnp.float32
    xb = x_ref[...]

    t = (jnp.dot(xb, qW1_ref[...], preferred_element_type=f32)
         + qb1_ref[...][None, :])
    t = _gelu(t)
    q = (jnp.dot(t, qW2_ref[...], preferred_element_type=f32)
         + qb2_ref[...][None, :])
    q = q * (1.0 / temp_ref[0])

    al = jax.lax.dot_general(q, keys_ref[...], (((1,), (1,)), ((), ())),
                             preferred_element_type=f32)
    al = al - jnp.max(al, axis=-1, keepdims=True)
    ea = jnp.exp(al)
    r = jnp.dot(ea, s2s_ref[...], preferred_element_type=f32)
    sl = r / jnp.sum(ea, axis=-1, keepdims=True)
    sl = sl - jnp.max(sl, axis=-1, keepdims=True)
    es = jnp.exp(sl)
    p = es / jnp.sum(es, axis=-1, keepdims=True)

    h = (jnp.dot(xb, w1_ref[:_IN, :], preferred_element_type=f32)
         + jnp.dot(ctx_ref[...], w1_ref[_IN:, :], preferred_element_type=f32)
         + hb1_ref[...])
    h = _gelu(h)

    eid = jax.lax.broadcasted_iota(jnp.int32, (_NSUB, _NSUB * _HH), 1) // _HH
    row = jax.lax.broadcasted_iota(jnp.int32, (_NSUB, _NSUB * _HH), 0)
    expand = (eid == row).astype(f32)
    pexp = jnp.dot(p, expand, preferred_element_type=f32)

    out = jnp.dot(h * pexp, w2_ref[...], preferred_element_type=f32)
    out = out + jnp.dot(p, hb2_ref[...], preferred_element_type=f32)
    out_ref[...] = out


def kernel(x, ctx, sys2sub, keys_p, qW1, qb1, qW2, qb2, hW1, hb1, hW2, hb2,
           temp):
    f32 = jnp.float32
    w1 = hW1.transpose(1, 0, 2).reshape(_IN + _CTX, _NSUB * _HH)
    w2 = hW2.reshape(_NSUB * _HH, _HO)
    b1 = hb1.reshape(1, _NSUB * _HH)

    grid = (_B // _BB,)
    tok = lambda i: (i, 0)
    rep = lambda i: (0, 0)

    return pl.pallas_call(
        _fused,
        grid=grid,
        in_specs=[
            pl.BlockSpec((_BB, _IN), tok),
            pl.BlockSpec((_BB, _CTX), tok),
            pl.BlockSpec((_NS, _NSUB), rep),
            pl.BlockSpec((_NS, _KD), rep),
            pl.BlockSpec((_IN, _KD), rep),
            pl.BlockSpec((_KD,), lambda i: (0,)),
            pl.BlockSpec((_KD, _KD), rep),
            pl.BlockSpec((_KD,), lambda i: (0,)),
            pl.BlockSpec((_IN + _CTX, _NSUB * _HH), rep),
            pl.BlockSpec((1, _NSUB * _HH), rep),
            pl.BlockSpec((_NSUB * _HH, _HO), rep),
            pl.BlockSpec((_NSUB, _HO), rep),
            pl.BlockSpec(memory_space=pltpu.SMEM),
        ],
        out_specs=pl.BlockSpec((_BB, _HO), tok),
        out_shape=jax.ShapeDtypeStruct((_B, _HO), f32),
        compiler_params=pltpu.CompilerParams(
            dimension_semantics=("parallel",)),
    )(x, ctx, sys2sub, keys_p, qW1, qb1, qW2, qb2, w1, b1, w2, hb2,
      temp.reshape(1))
